# baseline (speedup 1.0000x reference)
import functools

import jax
import jax.numpy as jnp
from jax import lax
from jax.experimental import pallas as pl
from jax.experimental.pallas import tpu as pltpu
from jax.experimental.pallas import tpu_sc as plsc

N = 10000
E = 640000
F0 = 112
DIM = 32
G = 128

NC, NS = 2, 16
LANES = 16
CH = 20000

@functools.cache
def _make_sc_agg(ft, n_fg, esplit):
    e_per = E // esplit
    nchunks = e_per // CH

    def body(tab_hbm, ep_hbm, out_hbm, tab, acc, ebuf0, ebuf1, sem0, sem1):
        wid = lax.axis_index("s") * NC + lax.axis_index("c")
        fg = wid % n_fg
        es = wid // n_fg

        @pl.when(wid < n_fg * esplit)
        def _():
            pltpu.sync_copy(tab_hbm.at[pl.ds(fg * ft, ft), :], tab)

            def zero_body(i, carry):
                for k in range(ft):
                    acc[k, pl.ds(i * LANES, LANES)] = jnp.zeros(
                        (LANES,), jnp.float32)
                return carry
            lax.fori_loop(0, N // LANES, zero_body, 0)

            ebase = es * e_per

            def consume(ebuf):
                @plsc.parallel_loop(0, CH // LANES, 1, unroll=16)
                def _groups(g2):
                    v = ebuf[pl.ds(g2 * LANES, LANES)]
                    s = v & 0xFFFF
                    d = lax.shift_right_logical(v, 16)
                    for k in range(ft):
                        kv = jnp.full((LANES,), k, jnp.int32)
                        plsc.addupdate_scatter(
                            acc, [kv, d], plsc.load_gather(tab, [kv, s]))

            def chunk_at(c):
                return ep_hbm.at[pl.ds(ebase + c * CH, CH)]

            pltpu.async_copy(chunk_at(0), ebuf0, sem0)

            def chunk_body(c2, carry):
                c = c2 * 2
                pltpu.async_copy(chunk_at(c + 1), ebuf1, sem1)
                pltpu.make_async_copy(chunk_at(c), ebuf0, sem0).wait()
                consume(ebuf0)

                @pl.when(c2 < nchunks // 2 - 1)
                def _():
                    pltpu.async_copy(chunk_at(c + 2), ebuf0, sem0)
                pltpu.make_async_copy(chunk_at(c + 1), ebuf1, sem1).wait()
                consume(ebuf1)
                return carry
            lax.fori_loop(0, nchunks // 2, chunk_body, 0)

            pltpu.sync_copy(acc, out_hbm.at[es, fg])

    mesh = plsc.VectorSubcoreMesh(
        core_axis_name="c", subcore_axis_name="s",
        num_cores=NC, num_subcores=NS)
    return pl.kernel(
        body,
        out_type=jax.ShapeDtypeStruct((esplit, n_fg, ft, N), jnp.float32),
        mesh=mesh,
        compiler_params=pltpu.CompilerParams(
            use_tc_tiling_on_sc=False, needs_layout_passes=False),
        scratch_types=[
            pltpu.VMEM((ft, N), jnp.float32),
            pltpu.VMEM((ft, N), jnp.float32),
            pltpu.VMEM((CH,), jnp.int32),
            pltpu.VMEM((CH,), jnp.int32),
            pltpu.SemaphoreType.DMA,
            pltpu.SemaphoreType.DMA,
        ],
    )


def _prep_body(x_ref, e_ref, xT_ref, ep_ref):
    xT_ref[...] = x_ref[...].T
    s = e_ref[0, :]
    d = e_ref[1, :]
    ep_ref[...] = jnp.bitwise_or(s, d << 16)


_prep = pl.pallas_call(
    _prep_body,
    out_shape=(jax.ShapeDtypeStruct((F0, N), jnp.float32),
               jax.ShapeDtypeStruct((E,), jnp.int32)),
)


def _dot_bf16(a, b):
    return jnp.dot(a.astype(jnp.bfloat16), b.astype(jnp.bfloat16),
                   preferred_element_type=jnp.float32)


def _bn_t(h, g_col, be_col):
    m = jnp.mean(h, axis=1, keepdims=True)
    var = jnp.mean((h - m) ** 2, axis=1, keepdims=True)
    return (h - m) * lax.rsqrt(var + 1e-5) * g_col + be_col


def _mlp1_body(xT_ref, aggT_ref, W1T_ref, b1_ref, W2T_ref, b2_ref,
               g1_ref, be1_ref, hT_ref):
    u = xT_ref[...] + aggT_ref[...]
    h = _dot_bf16(W1T_ref[...], u)
    h = jnp.maximum(h + b1_ref[...], 0.0)
    h = _dot_bf16(W2T_ref[...], h)
    h = jnp.maximum(h + b2_ref[...], 0.0)
    hT_ref[...] = _bn_t(h, g1_ref[...], be1_ref[...])


_mlp1 = pl.pallas_call(
    _mlp1_body,
    out_shape=jax.ShapeDtypeStruct((F0, N), jnp.float32),
)


def _tail_body(hT_ref, agg2_ref, W3T_ref, b3_ref, W4T_ref, b4_ref,
               g2_ref, be2_ref,
               batch_ref, WxdT_ref, bxd_ref, Wf1T_ref, bf1_ref,
               Wf2T_ref, bf2_ref, Wf3T_ref, bf3_ref, Wf4T_ref, bf4_ref,
               Wf5T_ref, bf5_ref, out_ref):
    h2in = hT_ref[...] + agg2_ref[...]
    q = jnp.maximum(_dot_bf16(W3T_ref[...], h2in) + b3_ref[...], 0.0)
    h2 = _dot_bf16(W4T_ref[...], q)
    h2 = jnp.maximum(h2 + b4_ref[...], 0.0)
    h2 = _bn_t(h2, g2_ref[...], be2_ref[...])
    seg = lax.broadcasted_iota(jnp.int32, (N, G), 1)
    P = (batch_ref[...] == seg).astype(jnp.float32)
    pooledT = jnp.dot(h2, P, preferred_element_type=jnp.float32,
                precision=lax.Precision.HIGHEST)
    z = _dot_bf16(WxdT_ref[...], pooledT)
    z = jnp.maximum(z + bxd_ref[...], 0.0)
    z = _dot_bf16(Wf1T_ref[...], z) + bf1_ref[...]
    z = _dot_bf16(Wf2T_ref[...], z) + bf2_ref[...]
    z = _dot_bf16(Wf3T_ref[...], z) + bf3_ref[...]
    z = _dot_bf16(Wf4T_ref[...], z) + bf4_ref[...]
    z = _dot_bf16(Wf5T_ref[...], z) + bf5_ref[...]
    out_ref[...] = z


_tail = pl.pallas_call(
    _tail_body,
    out_shape=jax.ShapeDtypeStruct((1, G), jnp.float32),
)


def kernel(x, edge_index, batch, W1, b1, W2, b2, g1, be1, W3, b3, W4, b4,
           g2, be2, Wxd, bxd, Wf1, bf1, Wf2, bf2, Wf3, bf3, Wf4, bf4,
           Wf5, bf5):
    col = lambda v: v.reshape(-1, 1)
    xT, ep = _prep(x, edge_index)
    agg1T = _make_sc_agg(4, 28, 1)(xT, ep).reshape(F0, N)
    hT = _mlp1(xT, agg1T, W1.T, col(b1), W2.T, col(b2), col(g1), col(be1))
    agg2T = _make_sc_agg(4, 28, 1)(hT, ep).reshape(F0, N)
    z = _tail(hT, agg2T, W3.T, col(b3), W4.T, col(b4), col(g2), col(be2),
              batch.reshape(N, 1), Wxd.T, col(bxd), Wf1.T, col(bf1),
              Wf2.T, col(bf2), Wf3.T, col(bf3), Wf4.T, col(bf4),
              Wf5.T, col(bf5))
    return z.reshape(G, 1)


# speedup vs baseline: 1.0058x; 1.0058x over previous
import functools

import jax
import jax.numpy as jnp
from jax import lax
from jax.experimental import pallas as pl
from jax.experimental.pallas import tpu as pltpu
from jax.experimental.pallas import tpu_sc as plsc

N = 10000
E = 640000
F0 = 112
DIM = 32
G = 128

NC, NS = 2, 16
LANES = 16
CH = 20000

@functools.cache
def _make_sc_agg(ft, n_fg, esplit):
    e_per = E // esplit
    nchunks = e_per // CH

    def body(tab_hbm, ep_hbm, out_hbm, tab, acc, ebuf0, ebuf1, sem0, sem1):
        wid = lax.axis_index("s") * NC + lax.axis_index("c")
        fg = wid % n_fg
        es = wid // n_fg

        @pl.when(wid < n_fg * esplit)
        def _():
            pltpu.sync_copy(tab_hbm.at[pl.ds(fg * ft, ft), :], tab)

            def zero_body(i, carry):
                for k in range(ft):
                    acc[k, pl.ds(i * LANES, LANES)] = jnp.zeros(
                        (LANES,), jnp.float32)
                return carry
            lax.fori_loop(0, N // LANES, zero_body, 0)

            ebase = es * e_per

            def consume(ebuf):
                @plsc.parallel_loop(0, CH // LANES, 1, unroll=8)
                def _groups(g2):
                    v = ebuf[pl.ds(g2 * LANES, LANES)]
                    s = v & 0xFFFF
                    d = lax.shift_right_logical(v, 16)
                    for k in range(ft):
                        kv = jnp.full((LANES,), k, jnp.int32)
                        plsc.addupdate_scatter(
                            acc, [kv, d], plsc.load_gather(tab, [kv, s]))

            def chunk_at(c):
                return ep_hbm.at[pl.ds(ebase + c * CH, CH)]

            pltpu.async_copy(chunk_at(0), ebuf0, sem0)

            def chunk_body(c2, carry):
                c = c2 * 2
                pltpu.async_copy(chunk_at(c + 1), ebuf1, sem1)
                pltpu.make_async_copy(chunk_at(c), ebuf0, sem0).wait()
                consume(ebuf0)

                @pl.when(c2 < nchunks // 2 - 1)
                def _():
                    pltpu.async_copy(chunk_at(c + 2), ebuf0, sem0)
                pltpu.make_async_copy(chunk_at(c + 1), ebuf1, sem1).wait()
                consume(ebuf1)
                return carry
            lax.fori_loop(0, nchunks // 2, chunk_body, 0)

            pltpu.sync_copy(acc, out_hbm.at[es, fg])

    mesh = plsc.VectorSubcoreMesh(
        core_axis_name="c", subcore_axis_name="s",
        num_cores=NC, num_subcores=NS)
    return pl.kernel(
        body,
        out_type=jax.ShapeDtypeStruct((esplit, n_fg, ft, N), jnp.float32),
        mesh=mesh,
        compiler_params=pltpu.CompilerParams(
            use_tc_tiling_on_sc=False, needs_layout_passes=False),
        scratch_types=[
            pltpu.VMEM((ft, N), jnp.float32),
            pltpu.VMEM((ft, N), jnp.float32),
            pltpu.VMEM((CH,), jnp.int32),
            pltpu.VMEM((CH,), jnp.int32),
            pltpu.SemaphoreType.DMA,
            pltpu.SemaphoreType.DMA,
        ],
    )


def _prep_body(x_ref, e_ref, xT_ref, ep_ref):
    xT_ref[...] = x_ref[...].T
    s = e_ref[0, :]
    d = e_ref[1, :]
    ep_ref[...] = jnp.bitwise_or(s, d << 16)


_prep = pl.pallas_call(
    _prep_body,
    out_shape=(jax.ShapeDtypeStruct((F0, N), jnp.float32),
               jax.ShapeDtypeStruct((E,), jnp.int32)),
)


def _dot_bf16(a, b):
    return jnp.dot(a.astype(jnp.bfloat16), b.astype(jnp.bfloat16),
                   preferred_element_type=jnp.float32)


def _bn_t(h, g_col, be_col):
    m = jnp.mean(h, axis=1, keepdims=True)
    var = jnp.mean((h - m) ** 2, axis=1, keepdims=True)
    return (h - m) * lax.rsqrt(var + 1e-5) * g_col + be_col


def _mlp1_body(xT_ref, aggT_ref, W1T_ref, b1_ref, W2T_ref, b2_ref,
               g1_ref, be1_ref, hT_ref):
    u = xT_ref[...] + aggT_ref[...]
    h = _dot_bf16(W1T_ref[...], u)
    h = jnp.maximum(h + b1_ref[...], 0.0)
    h = _dot_bf16(W2T_ref[...], h)
    h = jnp.maximum(h + b2_ref[...], 0.0)
    hT_ref[...] = _bn_t(h, g1_ref[...], be1_ref[...])


_mlp1 = pl.pallas_call(
    _mlp1_body,
    out_shape=jax.ShapeDtypeStruct((F0, N), jnp.float32),
)


def _tail_body(hT_ref, agg2_ref, W3T_ref, b3_ref, W4T_ref, b4_ref,
               g2_ref, be2_ref,
               batch_ref, WxdT_ref, bxd_ref, Wf1T_ref, bf1_ref,
               Wf2T_ref, bf2_ref, Wf3T_ref, bf3_ref, Wf4T_ref, bf4_ref,
               Wf5T_ref, bf5_ref, out_ref):
    h2in = hT_ref[...] + agg2_ref[...]
    q = jnp.maximum(_dot_bf16(W3T_ref[...], h2in) + b3_ref[...], 0.0)
    h2 = _dot_bf16(W4T_ref[...], q)
    h2 = jnp.maximum(h2 + b4_ref[...], 0.0)
    h2 = _bn_t(h2, g2_ref[...], be2_ref[...])
    seg = lax.broadcasted_iota(jnp.int32, (N, G), 1)
    P = (batch_ref[...] == seg).astype(jnp.float32)
    pooledT = jnp.dot(h2, P, preferred_element_type=jnp.float32,
                precision=lax.Precision.HIGHEST)
    z = _dot_bf16(WxdT_ref[...], pooledT)
    z = jnp.maximum(z + bxd_ref[...], 0.0)
    z = _dot_bf16(Wf1T_ref[...], z) + bf1_ref[...]
    z = _dot_bf16(Wf2T_ref[...], z) + bf2_ref[...]
    z = _dot_bf16(Wf3T_ref[...], z) + bf3_ref[...]
    z = _dot_bf16(Wf4T_ref[...], z) + bf4_ref[...]
    z = _dot_bf16(Wf5T_ref[...], z) + bf5_ref[...]
    out_ref[...] = z


_tail = pl.pallas_call(
    _tail_body,
    out_shape=jax.ShapeDtypeStruct((1, G), jnp.float32),
)


def kernel(x, edge_index, batch, W1, b1, W2, b2, g1, be1, W3, b3, W4, b4,
           g2, be2, Wxd, bxd, Wf1, bf1, Wf2, bf2, Wf3, bf3, Wf4, bf4,
           Wf5, bf5):
    col = lambda v: v.reshape(-1, 1)
    xT, ep = _prep(x, edge_index)
    agg1T = _make_sc_agg(4, 28, 1)(xT, ep).reshape(F0, N)
    hT = _mlp1(xT, agg1T, W1.T, col(b1), W2.T, col(b2), col(g1), col(be1))
    agg2T = _make_sc_agg(4, 28, 1)(hT, ep).reshape(F0, N)
    z = _tail(hT, agg2T, W3.T, col(b3), W4.T, col(b4), col(g2), col(be2),
              batch.reshape(N, 1), Wxd.T, col(bxd), Wf1.T, col(bf1),
              Wf2.T, col(bf2), Wf3.T, col(bf3), Wf4.T, col(bf4),
              Wf5.T, col(bf5))
    return z.reshape(G, 1)


# speedup vs baseline: 1.0104x; 1.0046x over previous
import functools

import jax
import jax.numpy as jnp
from jax import lax
from jax.experimental import pallas as pl
from jax.experimental.pallas import tpu as pltpu
from jax.experimental.pallas import tpu_sc as plsc

N = 10000
E = 640000
F0 = 112
DIM = 32
G = 128

NC, NS = 2, 16
LANES = 16
CH = 16000

@functools.cache
def _make_sc_agg(ft, n_fg, esplit):
    e_per = E // esplit
    nchunks = e_per // CH

    def body(tab_hbm, ep_hbm, out_hbm, tab, acc, ebuf0, ebuf1, sem0, sem1):
        wid = lax.axis_index("s") * NC + lax.axis_index("c")
        fg = wid % n_fg
        es = wid // n_fg

        @pl.when(wid < n_fg * esplit)
        def _():
            pltpu.sync_copy(tab_hbm.at[pl.ds(fg * ft, ft), :], tab)

            def zero_body(i, carry):
                for k in range(ft):
                    acc[k, pl.ds(i * LANES, LANES)] = jnp.zeros(
                        (LANES,), jnp.float32)
                return carry
            lax.fori_loop(0, N // LANES, zero_body, 0)

            ebase = es * e_per

            def consume(ebuf):
                @plsc.parallel_loop(0, CH // LANES, 1, unroll=8)
                def _groups(g2):
                    v = ebuf[pl.ds(g2 * LANES, LANES)]
                    s = v & 0xFFFF
                    d = lax.shift_right_logical(v, 16)
                    for k in range(ft):
                        kv = jnp.full((LANES,), k, jnp.int32)
                        plsc.addupdate_scatter(
                            acc, [kv, d], plsc.load_gather(tab, [kv, s]))

            def chunk_at(c):
                return ep_hbm.at[pl.ds(ebase + c * CH, CH)]

            pltpu.async_copy(chunk_at(0), ebuf0, sem0)

            def chunk_body(c2, carry):
                c = c2 * 2
                pltpu.async_copy(chunk_at(c + 1), ebuf1, sem1)
                pltpu.make_async_copy(chunk_at(c), ebuf0, sem0).wait()
                consume(ebuf0)

                @pl.when(c2 < nchunks // 2 - 1)
                def _():
                    pltpu.async_copy(chunk_at(c + 2), ebuf0, sem0)
                pltpu.make_async_copy(chunk_at(c + 1), ebuf1, sem1).wait()
                consume(ebuf1)
                return carry
            lax.fori_loop(0, nchunks // 2, chunk_body, 0)

            pltpu.sync_copy(acc, out_hbm.at[es, fg])

    mesh = plsc.VectorSubcoreMesh(
        core_axis_name="c", subcore_axis_name="s",
        num_cores=NC, num_subcores=NS)
    return pl.kernel(
        body,
        out_type=jax.ShapeDtypeStruct((esplit, n_fg, ft, N), jnp.float32),
        mesh=mesh,
        compiler_params=pltpu.CompilerParams(
            use_tc_tiling_on_sc=False, needs_layout_passes=False),
        scratch_types=[
            pltpu.VMEM((ft, N), jnp.float32),
            pltpu.VMEM((ft, N), jnp.float32),
            pltpu.VMEM((CH,), jnp.int32),
            pltpu.VMEM((CH,), jnp.int32),
            pltpu.SemaphoreType.DMA,
            pltpu.SemaphoreType.DMA,
        ],
    )


def _prep_body(x_ref, e_ref, xT_ref, ep_ref):
    xT_ref[...] = x_ref[...].T
    s = e_ref[0, :]
    d = e_ref[1, :]
    ep_ref[...] = jnp.bitwise_or(s, d << 16)


_prep = pl.pallas_call(
    _prep_body,
    out_shape=(jax.ShapeDtypeStruct((F0, N), jnp.float32),
               jax.ShapeDtypeStruct((E,), jnp.int32)),
)


def _dot_bf16(a, b):
    return jnp.dot(a.astype(jnp.bfloat16), b.astype(jnp.bfloat16),
                   preferred_element_type=jnp.float32)


def _bn_t(h, g_col, be_col):
    m = jnp.mean(h, axis=1, keepdims=True)
    var = jnp.mean((h - m) ** 2, axis=1, keepdims=True)
    return (h - m) * lax.rsqrt(var + 1e-5) * g_col + be_col


def _mlp1_body(xT_ref, aggT_ref, W1T_ref, b1_ref, W2T_ref, b2_ref,
               g1_ref, be1_ref, hT_ref):
    u = xT_ref[...] + aggT_ref[...]
    h = _dot_bf16(W1T_ref[...], u)
    h = jnp.maximum(h + b1_ref[...], 0.0)
    h = _dot_bf16(W2T_ref[...], h)
    h = jnp.maximum(h + b2_ref[...], 0.0)
    hT_ref[...] = _bn_t(h, g1_ref[...], be1_ref[...])


_mlp1 = pl.pallas_call(
    _mlp1_body,
    out_shape=jax.ShapeDtypeStruct((F0, N), jnp.float32),
)


def _tail_body(hT_ref, agg2_ref, W3T_ref, b3_ref, W4T_ref, b4_ref,
               g2_ref, be2_ref,
               batch_ref, WxdT_ref, bxd_ref, Wf1T_ref, bf1_ref,
               Wf2T_ref, bf2_ref, Wf3T_ref, bf3_ref, Wf4T_ref, bf4_ref,
               Wf5T_ref, bf5_ref, out_ref):
    h2in = hT_ref[...] + agg2_ref[...]
    q = jnp.maximum(_dot_bf16(W3T_ref[...], h2in) + b3_ref[...], 0.0)
    h2 = _dot_bf16(W4T_ref[...], q)
    h2 = jnp.maximum(h2 + b4_ref[...], 0.0)
    h2 = _bn_t(h2, g2_ref[...], be2_ref[...])
    seg = lax.broadcasted_iota(jnp.int32, (N, G), 1)
    P = (batch_ref[...] == seg).astype(jnp.float32)
    pooledT = jnp.dot(h2, P, preferred_element_type=jnp.float32,
                precision=lax.Precision.HIGHEST)
    z = _dot_bf16(WxdT_ref[...], pooledT)
    z = jnp.maximum(z + bxd_ref[...], 0.0)
    z = _dot_bf16(Wf1T_ref[...], z) + bf1_ref[...]
    z = _dot_bf16(Wf2T_ref[...], z) + bf2_ref[...]
    z = _dot_bf16(Wf3T_ref[...], z) + bf3_ref[...]
    z = _dot_bf16(Wf4T_ref[...], z) + bf4_ref[...]
    z = _dot_bf16(Wf5T_ref[...], z) + bf5_ref[...]
    out_ref[...] = z


_tail = pl.pallas_call(
    _tail_body,
    out_shape=jax.ShapeDtypeStruct((1, G), jnp.float32),
)


def kernel(x, edge_index, batch, W1, b1, W2, b2, g1, be1, W3, b3, W4, b4,
           g2, be2, Wxd, bxd, Wf1, bf1, Wf2, bf2, Wf3, bf3, Wf4, bf4,
           Wf5, bf5):
    col = lambda v: v.reshape(-1, 1)
    xT, ep = _prep(x, edge_index)
    agg1T = _make_sc_agg(4, 28, 1)(xT, ep).reshape(F0, N)
    hT = _mlp1(xT, agg1T, W1.T, col(b1), W2.T, col(b2), col(g1), col(be1))
    agg2T = _make_sc_agg(4, 28, 1)(hT, ep).reshape(F0, N)
    z = _tail(hT, agg2T, W3.T, col(b3), W4.T, col(b4), col(g2), col(be2),
              batch.reshape(N, 1), Wxd.T, col(bxd), Wf1.T, col(bf1),
              Wf2.T, col(bf2), Wf3.T, col(bf3), Wf4.T, col(bf4),
              Wf5.T, col(bf5))
    return z.reshape(G, 1)
